# Initial kernel scaffold; baseline (speedup 1.0000x reference)
import jax
import jax.numpy as jnp
from jax.experimental import pallas as pl


def kernel(pitch, program, velocity, pitch_table, program_table, velocity_table):
    raise NotImplementedError("write your pallas kernel here")


# speedup vs baseline: 2.5327x; 2.5327x over previous
import functools

import jax
import jax.numpy as jnp
from jax import lax
from jax.experimental import pallas as pl
from jax.experimental.pallas import tpu as pltpu
from jax.experimental.pallas import tpu_sc as plsc

D = 512
CHUNK = 64


@functools.cache
def _make_kernel(N: int):
    info = plsc.get_sparse_core_info()
    NC, NS = info.num_cores, info.num_subcores
    NW = NC * NS
    TPW = N // NW
    n_chunks = TPW // CHUNK
    mesh = plsc.VectorSubcoreMesh(core_axis_name="c", subcore_axis_name="s")

    @functools.partial(
        pl.kernel,
        mesh=mesh,
        out_type=jax.ShapeDtypeStruct((N, 3 * D), jnp.float32),
        scratch_types=[
            pltpu.VMEM((CHUNK,), jnp.int32),
            pltpu.VMEM((CHUNK,), jnp.int32),
            pltpu.VMEM((CHUNK,), jnp.int32),
            pltpu.VMEM((CHUNK, D), jnp.float32),
            pltpu.VMEM((CHUNK, D), jnp.float32),
            pltpu.VMEM((CHUNK, D), jnp.float32),
            pltpu.SemaphoreType.DMA,
        ],
    )
    def k(pitch_h, program_h, velocity_h, ptab_h, gtab_h, vtab_h, out_h,
          pidx, gidx, vidx, prow, grow, vrow, sem):
        wid = lax.axis_index("s") * NC + lax.axis_index("c")
        base = wid * TPW

        def body(i, carry):
            off = base + i * CHUNK
            pltpu.sync_copy(pitch_h.at[pl.ds(off, CHUNK)], pidx)
            pltpu.sync_copy(program_h.at[pl.ds(off, CHUNK)], gidx)
            pltpu.sync_copy(velocity_h.at[pl.ds(off, CHUNK)], vidx)
            cp = pltpu.async_copy(ptab_h.at[pidx], prow, sem)
            cg = pltpu.async_copy(gtab_h.at[gidx], grow, sem)
            cv = pltpu.async_copy(vtab_h.at[vidx], vrow, sem)
            cp.wait()
            cg.wait()
            cv.wait()
            pltpu.sync_copy(prow, out_h.at[pl.ds(off, CHUNK), pl.ds(0, D)])
            pltpu.sync_copy(grow, out_h.at[pl.ds(off, CHUNK), pl.ds(D, D)])
            pltpu.sync_copy(vrow, out_h.at[pl.ds(off, CHUNK), pl.ds(2 * D, D)])
            return carry

        lax.fori_loop(0, n_chunks, body, 0)

    return k


def kernel(pitch, program, velocity, pitch_table, program_table, velocity_table):
    B, S = pitch.shape
    N = B * S
    p = pitch.reshape(N).astype(jnp.int32)
    g = program.reshape(N).astype(jnp.int32)
    v = velocity.reshape(N).astype(jnp.int32)
    out = _make_kernel(N)(p, g, v, pitch_table, program_table, velocity_table)
    return out.reshape(B, S, 3 * D)
